# baseline (speedup 1.0000x reference)
import functools

import jax
import jax.numpy as jnp
import numpy as np
from jax import lax
from jax.experimental import pallas as pl
from jax.experimental.pallas import tpu as pltpu
from jax.experimental.pallas import tpu_sc as plsc

N = 10000
E = 320000
D_NODE = 128
D_EDGE = 16
HID = 16
OUT = 128

NC = 2
NS = 16
NW = NC * NS
B = 80
EPT = 10240
E_PAD = NW * EPT
CHUNKS = EPT // B
N_PAD = 10112
ROWS_PER_TILE = N_PAD // NS
PAIRS = 9
TCOLS = PAIRS * 32
NBUF = 2
STG = 624


def _mm_body(out_dtype, x_ref, w_ref, o_ref):
    o_ref[...] = jnp.dot(x_ref[...], w_ref[...],
                         preferred_element_type=jnp.float32).astype(out_dtype)


def _tc_matmul(x, w, out_dtype=jnp.float32):
    return pl.pallas_call(
        functools.partial(_mm_body, out_dtype),
        out_shape=jax.ShapeDtypeStruct((x.shape[0], w.shape[1]), out_dtype),
    )(x, w)


def _combine_body(make_t2, agg_ref, x_ref, root_ref, bias_ref, gamma_ref,
                  beta_ref, w2_ref, h_ref, t2_ref=None):
    agg = agg_ref[0, :N, :] + agg_ref[1, :N, :]
    h = agg + jnp.dot(x_ref[...], root_ref[...],
                      preferred_element_type=jnp.float32) + bias_ref[...]
    mean = jnp.mean(h, axis=0, keepdims=True)
    d = h - mean
    var = jnp.mean(d * d, axis=0, keepdims=True)
    hn = d * lax.rsqrt(var + 1e-5) * gamma_ref[...] + beta_ref[...]
    hact = jnp.where(hn > 0, hn, 0.01 * hn)
    h_ref[...] = hact
    if make_t2:
        t2_ref[...] = jnp.dot(hact, w2_ref[...],
                              preferred_element_type=jnp.float32
                              ).astype(jnp.bfloat16)


def _tc_combine(agg, x, root, bias, gamma, beta, w2, make_t2):
    out_shape = [jax.ShapeDtypeStruct((N, root.shape[1]), jnp.float32)]
    if make_t2:
        out_shape.append(jax.ShapeDtypeStruct((N, w2.shape[1]), jnp.bfloat16))
    res = pl.pallas_call(
        functools.partial(_combine_body, make_t2),
        out_shape=out_shape,
    )(agg, x, root, bias, gamma, beta, w2)
    return res


_MESH = plsc.VectorSubcoreMesh(core_axis_name="c", subcore_axis_name="s")

_GDN = lax.GatherDimensionNumbers(
    offset_dims=(), collapsed_slice_dims=(0,), start_index_map=(0,))


def _lane_bcast(vec, k):
    idx = jnp.full((16, 1), k, jnp.int32)
    return lax.gather(vec, idx, _GDN, slice_sizes=(1,),
                      mode=lax.GatherScatterMode.PROMISE_IN_BOUNDS)


def _lo(v):
    return lax.bitcast_convert_type(lax.shift_left(v, 16), jnp.float32)


def _hi(v):
    return lax.bitcast_convert_type(v, jnp.float32)


@functools.partial(
    pl.kernel,
    out_type=jax.ShapeDtypeStruct((NC, N_PAD, HID), jnp.float32),
    mesh=_MESH,
    scratch_types=[
        pltpu.VMEM((NBUF, 2, B), jnp.int32),
        pltpu.VMEM((NBUF, B, D_EDGE), jnp.float32),
        pltpu.VMEM((NBUF, B, TCOLS // 2), jnp.int32),
        pltpu.VMEM((B, HID), jnp.float32),
        pltpu.VMEM_SHARED((N, TCOLS // 2), jnp.int32),
        pltpu.VMEM_SHARED((N_PAD, HID), jnp.float32),
        pltpu.SemaphoreType.DMA((NBUF, 4)),
        pltpu.SemaphoreType.DMA((NBUF,)),
        pltpu.SemaphoreType.DMA((NBUF,)),
    ],
    compiler_params=pltpu.CompilerParams(use_tc_tiling_on_sc=False),
)
def _edge_pass(table_hbm, sd_hbm, ef_hbm, zero_hbm, out_hbm,
               sd_v, ef_v, rows_v, msg_v, tbl_sh, agg_sh, gsem, esem, isem):
    c = lax.axis_index("c")
    s = lax.axis_index("s")
    wid = s * NC + c

    pltpu.sync_copy(zero_hbm, agg_sh.at[pl.ds(s * ROWS_PER_TILE,
                                              ROWS_PER_TILE)])
    pltpu.sync_copy(table_hbm.at[pl.ds(s * STG, STG)],
                    tbl_sh.at[pl.ds(s * STG, STG)])

    @pl.when(s == 0)
    def _():
        pltpu.sync_copy(table_hbm.at[pl.ds(NS * STG, N - NS * STG)],
                        tbl_sh.at[pl.ds(NS * STG, N - NS * STG)])

    plsc.subcore_barrier()

    def start_idx(j, slot):
        pltpu.async_copy(sd_hbm.at[wid * CHUNKS + j], sd_v.at[slot],
                         isem.at[slot])

    def wait_idx(j, slot):
        pltpu.make_async_copy(sd_hbm.at[wid * CHUNKS + j], sd_v.at[slot],
                              isem.at[slot]).wait()

    SPLITS = ((0, 24), (24, 24), (48, 16), (64, 16))

    def start_gather(j, slot):
        e0 = jnp.minimum((wid * CHUNKS + j) * B, E - B)
        pltpu.async_copy(ef_hbm.at[pl.ds(e0, B)], ef_v.at[slot],
                         esem.at[slot])
        for h, (o, n) in enumerate(SPLITS):
            pltpu.async_copy(tbl_sh.at[sd_v.at[slot, 0, pl.ds(o, n)]],
                             rows_v.at[slot, pl.ds(o, n)],
                             gsem.at[slot, h])

    def wait_gather(j, slot):
        e0 = jnp.minimum((wid * CHUNKS + j) * B, E - B)
        pltpu.make_async_copy(ef_hbm.at[pl.ds(e0, B)], ef_v.at[slot],
                              esem.at[slot]).wait()
        for h, (o, n) in enumerate(SPLITS):
            pltpu.make_async_copy(tbl_sh.at[sd_v.at[slot, 0, pl.ds(o, n)]],
                                  rows_v.at[slot, pl.ds(o, n)],
                                  gsem.at[slot, h]).wait()

    def process(j, slot):
        def _one_edge(b):
            efr = ef_v[slot, b, :]
            paccs = [rows_v[slot, b, pl.ds(8 * 16, 16)], None, None, None]
            paccs[0] = _lo(paccs[0])
            for p in range(8):
                pair = rows_v[slot, b, pl.ds(p * 16, 16)]
                lane = p % 4
                term = (_lane_bcast(efr, 2 * p) * _lo(pair)
                        + _lane_bcast(efr, 2 * p + 1) * _hi(pair))
                if paccs[lane] is None:
                    paccs[lane] = term
                else:
                    paccs[lane] = paccs[lane] + term
            msg_v[b, :] = (paccs[0] + paccs[1]) + (paccs[2] + paccs[3])

        def _body(bb, carry2):
            _one_edge(2 * bb)
            _one_edge(2 * bb + 1)
            return carry2

        lax.fori_loop(0, B // 2, _body, 0)
        pltpu.sync_copy(msg_v, agg_sh.at[sd_v.at[slot, 1]], add=True)

    start_idx(0, 0)
    wait_idx(0, 0)
    start_gather(0, 0)
    start_idx(1, 1)

    def outer(jj, carry):
        for slot in range(NBUF):
            j = jj * NBUF + slot
            nslot = (slot + 1) % NBUF

            @pl.when(j + 1 < CHUNKS)
            def _():
                wait_idx(j + 1, nslot)
                start_gather(j + 1, nslot)

            wait_gather(j, slot)
            process(j, slot)

            @pl.when(j + 2 < CHUNKS)
            def _():
                start_idx(j + 2, slot)
        return carry

    lax.fori_loop(0, CHUNKS // NBUF, outer, 0)
    plsc.subcore_barrier()
    pltpu.sync_copy(agg_sh.at[pl.ds(s * ROWS_PER_TILE, ROWS_PER_TILE)],
                    out_hbm.at[c, pl.ds(s * ROWS_PER_TILE, ROWS_PER_TILE)])


_PERM = np.zeros((TCOLS,), np.int32)
for _p in range(PAIRS):
    for _o in range(16):
        for _r in range(2):
            _PERM[_p * 32 + 2 * _o + _r] = (2 * _p + _r) * 16 + _o


def _pack_wcat(We, be, in_c, out_c):
    Wr = We.reshape(in_c, out_c, D_EDGE).transpose(0, 2, 1)
    Wr = Wr.reshape(in_c, D_EDGE * out_c)
    br = be.reshape(in_c, out_c)
    wcat = jnp.concatenate(
        [Wr, br, jnp.zeros((in_c, 16), jnp.float32)], axis=1)
    return wcat[:, _PERM]


@jax.jit
def kernel(node_features, edge_index, edge_features, batch,
           We1, be1, root1, bias1, gamma1, beta1,
           We2, be2, root2, bias2, gamma2, beta2,
           Wl, bl):
    x = node_features
    src = edge_index[0]
    dst = edge_index[1]

    pad = E_PAD - E
    src_p = jnp.concatenate([src, jnp.zeros((pad,), jnp.int32)])
    dst_p = jnp.concatenate([dst, jnp.full((pad,), N, jnp.int32)])
    sd = jnp.stack([src_p.reshape(E_PAD // B, B),
                    dst_p.reshape(E_PAD // B, B)], axis=1)
    zero_rows = jnp.zeros((ROWS_PER_TILE, HID), jnp.float32)

    wcat1 = _pack_wcat(We1, be1, D_NODE, HID)
    wcat2 = _pack_wcat(We2, be2, HID, HID)

    t1 = _tc_matmul(x, wcat1, jnp.bfloat16)
    t1i = lax.bitcast_convert_type(
        t1.reshape(N, TCOLS // 2, 2), jnp.int32)
    agg1 = _edge_pass(t1i, sd, edge_features, zero_rows)
    h1, t2 = _tc_combine(agg1, x, root1, bias1.reshape(1, HID),
                         gamma1.reshape(1, HID), beta1.reshape(1, HID),
                         wcat2, True)
    t2i = lax.bitcast_convert_type(
        t2.reshape(N, TCOLS // 2, 2), jnp.int32)
    agg2 = _edge_pass(t2i, sd, edge_features, zero_rows)
    (h2,) = _tc_combine(agg2, h1, root2, bias2.reshape(1, HID),
                        gamma2.reshape(1, HID), beta2.reshape(1, HID),
                        wcat2, False)
    out = _tc_matmul(h2, Wl.T) + bl.reshape(1, OUT)
    return out


# speedup vs baseline: 1.4770x; 1.4770x over previous
import functools

import jax
import jax.numpy as jnp
import numpy as np
from jax import lax
from jax.experimental import pallas as pl
from jax.experimental.pallas import tpu as pltpu
from jax.experimental.pallas import tpu_sc as plsc

N = 10000
E = 320000
D_NODE = 128
D_EDGE = 16
HID = 16
OUT = 128

NC = 2
NS = 16
NW = NC * NS
B = 80
EPT = 10240
E_PAD = NW * EPT
CHUNKS = EPT // B
N_PAD = 10112
ROWS_PER_TILE = N_PAD // NS
PAIRS = 9
TCOLS = PAIRS * 32
NBUF = 2
STG = 624


def _pack_i32(t):
    a = lax.bitcast_convert_type(t[:, :TCOLS // 2].astype(jnp.bfloat16),
                                 jnp.uint16).astype(jnp.int32)
    b = lax.bitcast_convert_type(t[:, TCOLS // 2:].astype(jnp.bfloat16),
                                 jnp.uint16).astype(jnp.int32)
    return lax.bitwise_or(a, lax.shift_left(b, 16))


def _mm_body(pack, x_ref, w_ref, o_ref):
    t = jnp.dot(x_ref[...], w_ref[...], preferred_element_type=jnp.float32)
    o_ref[...] = _pack_i32(t) if pack else t


def _tc_matmul(x, w, pack=False):
    n_out = w.shape[1] // 2 if pack else w.shape[1]
    dt = jnp.int32 if pack else jnp.float32
    return pl.pallas_call(
        functools.partial(_mm_body, pack),
        out_shape=jax.ShapeDtypeStruct((x.shape[0], n_out), dt),
    )(x, w)


def _sd_body(ei_ref, o_ref):
    o_ref[0:1, :E] = ei_ref[0:1, :]
    o_ref[1:2, :E] = ei_ref[1:2, :]
    o_ref[0:1, E:] = jnp.zeros((1, E_PAD - E), jnp.int32)
    o_ref[1:2, E:] = jnp.full((1, E_PAD - E), N, jnp.int32)


def _tc_sd(edge_index):
    return pl.pallas_call(
        _sd_body,
        out_shape=jax.ShapeDtypeStruct((2, E_PAD), jnp.int32),
    )(edge_index)


def _combine_body(make_t2, agg_ref, x_ref, root_ref, bias_ref, gamma_ref,
                  beta_ref, w2_ref, h_ref, t2_ref=None):
    agg = agg_ref[0, :N, :] + agg_ref[1, :N, :]
    h = agg + jnp.dot(x_ref[...], root_ref[...],
                      preferred_element_type=jnp.float32) + bias_ref[...]
    mean = jnp.mean(h, axis=0, keepdims=True)
    d = h - mean
    var = jnp.mean(d * d, axis=0, keepdims=True)
    hn = d * lax.rsqrt(var + 1e-5) * gamma_ref[...] + beta_ref[...]
    hact = jnp.where(hn > 0, hn, 0.01 * hn)
    h_ref[...] = hact
    if make_t2:
        t2_ref[...] = _pack_i32(jnp.dot(hact, w2_ref[...],
                                        preferred_element_type=jnp.float32))


def _tc_combine(agg, x, root, bias, gamma, beta, w2, make_t2):
    out_shape = [jax.ShapeDtypeStruct((N, root.shape[1]), jnp.float32)]
    if make_t2:
        out_shape.append(
            jax.ShapeDtypeStruct((N, w2.shape[1] // 2), jnp.int32))
    res = pl.pallas_call(
        functools.partial(_combine_body, make_t2),
        out_shape=out_shape,
    )(agg, x, root, bias, gamma, beta, w2)
    return res


_MESH = plsc.VectorSubcoreMesh(core_axis_name="c", subcore_axis_name="s")

_GDN = lax.GatherDimensionNumbers(
    offset_dims=(), collapsed_slice_dims=(0,), start_index_map=(0,))


def _lane_bcast(vec, k):
    idx = jnp.full((16, 1), k, jnp.int32)
    return lax.gather(vec, idx, _GDN, slice_sizes=(1,),
                      mode=lax.GatherScatterMode.PROMISE_IN_BOUNDS)


def _lo(v):
    return lax.bitcast_convert_type(lax.shift_left(v, 16), jnp.float32)


def _hi(v):
    return lax.bitcast_convert_type(v, jnp.float32)


@functools.partial(
    pl.kernel,
    out_type=jax.ShapeDtypeStruct((NC, N_PAD, HID), jnp.float32),
    mesh=_MESH,
    scratch_types=[
        pltpu.VMEM((NBUF, 2, B), jnp.int32),
        pltpu.VMEM((NBUF, B, D_EDGE), jnp.float32),
        pltpu.VMEM((NBUF, B, TCOLS // 2), jnp.int32),
        pltpu.VMEM((B, HID), jnp.float32),
        pltpu.VMEM_SHARED((N, TCOLS // 2), jnp.int32),
        pltpu.VMEM_SHARED((N_PAD, HID), jnp.float32),
        pltpu.SemaphoreType.DMA((NBUF, 4)),
        pltpu.SemaphoreType.DMA((NBUF,)),
        pltpu.SemaphoreType.DMA((NBUF,)),
    ],
    compiler_params=pltpu.CompilerParams(use_tc_tiling_on_sc=False),
)
def _edge_pass(table_hbm, sd_hbm, ef_hbm, zero_hbm, out_hbm,
               sd_v, ef_v, rows_v, msg_v, tbl_sh, agg_sh, gsem, esem, isem):
    c = lax.axis_index("c")
    s = lax.axis_index("s")
    wid = s * NC + c

    pltpu.sync_copy(zero_hbm, agg_sh.at[pl.ds(s * ROWS_PER_TILE,
                                              ROWS_PER_TILE)])
    pltpu.sync_copy(table_hbm.at[pl.ds(s * STG, STG)],
                    tbl_sh.at[pl.ds(s * STG, STG)])

    @pl.when(s == 0)
    def _():
        pltpu.sync_copy(table_hbm.at[pl.ds(NS * STG, N - NS * STG)],
                        tbl_sh.at[pl.ds(NS * STG, N - NS * STG)])

    plsc.subcore_barrier()

    def start_idx(j, slot):
        e0 = (wid * CHUNKS + j) * B
        pltpu.async_copy(sd_hbm.at[:, pl.ds(e0, B)], sd_v.at[slot],
                         isem.at[slot])

    def wait_idx(j, slot):
        e0 = (wid * CHUNKS + j) * B
        pltpu.make_async_copy(sd_hbm.at[:, pl.ds(e0, B)], sd_v.at[slot],
                              isem.at[slot]).wait()

    SPLITS = ((0, 24), (24, 24), (48, 16), (64, 16))

    def start_gather(j, slot):
        e0 = jnp.minimum((wid * CHUNKS + j) * B, E - B)
        pltpu.async_copy(ef_hbm.at[pl.ds(e0, B)], ef_v.at[slot],
                         esem.at[slot])
        for h, (o, n) in enumerate(SPLITS):
            pltpu.async_copy(tbl_sh.at[sd_v.at[slot, 0, pl.ds(o, n)]],
                             rows_v.at[slot, pl.ds(o, n)],
                             gsem.at[slot, h])

    def wait_gather(j, slot):
        e0 = jnp.minimum((wid * CHUNKS + j) * B, E - B)
        pltpu.make_async_copy(ef_hbm.at[pl.ds(e0, B)], ef_v.at[slot],
                              esem.at[slot]).wait()
        for h, (o, n) in enumerate(SPLITS):
            pltpu.make_async_copy(tbl_sh.at[sd_v.at[slot, 0, pl.ds(o, n)]],
                                  rows_v.at[slot, pl.ds(o, n)],
                                  gsem.at[slot, h]).wait()

    def process(j, slot):
        def _one_edge(b):
            efr = ef_v[slot, b, :]
            paccs = [rows_v[slot, b, pl.ds(8 * 16, 16)], None, None, None]
            paccs[0] = _lo(paccs[0])
            for p in range(8):
                pair = rows_v[slot, b, pl.ds(p * 16, 16)]
                lane = p % 4
                term = (_lane_bcast(efr, 2 * p) * _lo(pair)
                        + _lane_bcast(efr, 2 * p + 1) * _hi(pair))
                if paccs[lane] is None:
                    paccs[lane] = term
                else:
                    paccs[lane] = paccs[lane] + term
            msg_v[b, :] = (paccs[0] + paccs[1]) + (paccs[2] + paccs[3])

        def _body(bb, carry2):
            _one_edge(2 * bb)
            _one_edge(2 * bb + 1)
            return carry2

        lax.fori_loop(0, B // 2, _body, 0)
        pltpu.sync_copy(msg_v, agg_sh.at[sd_v.at[slot, 1]], add=True)

    start_idx(0, 0)
    wait_idx(0, 0)
    start_gather(0, 0)
    start_idx(1, 1)

    def outer(jj, carry):
        for slot in range(NBUF):
            j = jj * NBUF + slot
            nslot = (slot + 1) % NBUF

            @pl.when(j + 1 < CHUNKS)
            def _():
                wait_idx(j + 1, nslot)
                start_gather(j + 1, nslot)

            wait_gather(j, slot)
            process(j, slot)

            @pl.when(j + 2 < CHUNKS)
            def _():
                start_idx(j + 2, slot)
        return carry

    lax.fori_loop(0, CHUNKS // NBUF, outer, 0)
    plsc.subcore_barrier()
    pltpu.sync_copy(agg_sh.at[pl.ds(s * ROWS_PER_TILE, ROWS_PER_TILE)],
                    out_hbm.at[c, pl.ds(s * ROWS_PER_TILE, ROWS_PER_TILE)])


_PERM = np.zeros((TCOLS,), np.int32)
for _p in range(PAIRS):
    for _o in range(16):
        _PERM[_p * 16 + _o] = (2 * _p) * 16 + _o
        _PERM[TCOLS // 2 + _p * 16 + _o] = (2 * _p + 1) * 16 + _o


def _pack_wcat(We, be, in_c, out_c):
    Wr = We.reshape(in_c, out_c, D_EDGE).transpose(0, 2, 1)
    Wr = Wr.reshape(in_c, D_EDGE * out_c)
    br = be.reshape(in_c, out_c)
    wcat = jnp.concatenate(
        [Wr, br, jnp.zeros((in_c, 16), jnp.float32)], axis=1)
    return wcat[:, _PERM]


@jax.jit
def kernel(node_features, edge_index, edge_features, batch,
           We1, be1, root1, bias1, gamma1, beta1,
           We2, be2, root2, bias2, gamma2, beta2,
           Wl, bl):
    x = node_features
    src = edge_index[0]
    dst = edge_index[1]

    sd = _tc_sd(edge_index)
    zero_rows = jnp.zeros((ROWS_PER_TILE, HID), jnp.float32)

    wcat1 = _pack_wcat(We1, be1, D_NODE, HID)
    wcat2 = _pack_wcat(We2, be2, HID, HID)

    t1i = _tc_matmul(x, wcat1, pack=True)
    agg1 = _edge_pass(t1i, sd, edge_features, zero_rows)
    h1, t2 = _tc_combine(agg1, x, root1, bias1.reshape(1, HID),
                         gamma1.reshape(1, HID), beta1.reshape(1, HID),
                         wcat2, True)
    agg2 = _edge_pass(t2, sd, edge_features, zero_rows)
    (h2,) = _tc_combine(agg2, h1, root2, bias2.reshape(1, HID),
                        gamma2.reshape(1, HID), beta2.reshape(1, HID),
                        wcat2, False)
    out = _tc_matmul(h2, Wl.T) + bl.reshape(1, OUT)
    return out


# speedup vs baseline: 1.4854x; 1.0057x over previous
import functools

import jax
import jax.numpy as jnp
import numpy as np
from jax import lax
from jax.experimental import pallas as pl
from jax.experimental.pallas import tpu as pltpu
from jax.experimental.pallas import tpu_sc as plsc

N = 10000
E = 320000
D_NODE = 128
D_EDGE = 16
HID = 16
OUT = 128

NC = 2
NS = 16
NW = NC * NS
B = 80
EPT = 10240
E_PAD = NW * EPT
CHUNKS = EPT // B
N_PAD = 10112
ROWS_PER_TILE = N_PAD // NS
PAIRS = 9
TCOLS = PAIRS * 32
NBUF = 2
STG = 624


def _pack_i32(t):
    a = lax.bitcast_convert_type(t[:, :TCOLS // 2].astype(jnp.bfloat16),
                                 jnp.uint16).astype(jnp.int32)
    b = lax.bitcast_convert_type(t[:, TCOLS // 2:].astype(jnp.bfloat16),
                                 jnp.uint16).astype(jnp.int32)
    return lax.bitwise_or(a, lax.shift_left(b, 16))


def _mm_body(pack, x_ref, w_ref, o_ref):
    t = jnp.dot(x_ref[...], w_ref[...], preferred_element_type=jnp.float32)
    o_ref[...] = _pack_i32(t) if pack else t


def _tc_matmul(x, w, pack=False):
    n_out = w.shape[1] // 2 if pack else w.shape[1]
    dt = jnp.int32 if pack else jnp.float32
    return pl.pallas_call(
        functools.partial(_mm_body, pack),
        out_shape=jax.ShapeDtypeStruct((x.shape[0], n_out), dt),
    )(x, w)


def _mmb_body(x_ref, w_ref, b_ref, o_ref):
    o_ref[...] = jnp.dot(x_ref[...], w_ref[...],
                         preferred_element_type=jnp.float32) + b_ref[...]


def _tc_matmul_bias(x, w, b):
    return pl.pallas_call(
        _mmb_body,
        out_shape=jax.ShapeDtypeStruct((x.shape[0], w.shape[1]), jnp.float32),
    )(x, w, b)


def _sd_body(ei_ref, o_ref):
    o_ref[0:1, :E] = ei_ref[0:1, :]
    o_ref[1:2, :E] = ei_ref[1:2, :]
    o_ref[0:1, E:] = jnp.zeros((1, E_PAD - E), jnp.int32)
    o_ref[1:2, E:] = jnp.full((1, E_PAD - E), N, jnp.int32)


def _tc_sd(edge_index):
    return pl.pallas_call(
        _sd_body,
        out_shape=jax.ShapeDtypeStruct((2, E_PAD), jnp.int32),
    )(edge_index)


def _combine_body(make_t2, agg_ref, x_ref, root_ref, bias_ref, gamma_ref,
                  beta_ref, w2_ref, h_ref, t2_ref=None):
    agg = agg_ref[0, :N, :] + agg_ref[1, :N, :]
    h = agg + jnp.dot(x_ref[...], root_ref[...],
                      preferred_element_type=jnp.float32) + bias_ref[...]
    mean = jnp.mean(h, axis=0, keepdims=True)
    d = h - mean
    var = jnp.mean(d * d, axis=0, keepdims=True)
    hn = d * lax.rsqrt(var + 1e-5) * gamma_ref[...] + beta_ref[...]
    hact = jnp.where(hn > 0, hn, 0.01 * hn)
    h_ref[...] = hact
    if make_t2:
        t2_ref[...] = _pack_i32(jnp.dot(hact, w2_ref[...],
                                        preferred_element_type=jnp.float32))


def _tc_combine(agg, x, root, bias, gamma, beta, w2, make_t2):
    out_shape = [jax.ShapeDtypeStruct((N, root.shape[1]), jnp.float32)]
    if make_t2:
        out_shape.append(
            jax.ShapeDtypeStruct((N, w2.shape[1] // 2), jnp.int32))
    res = pl.pallas_call(
        functools.partial(_combine_body, make_t2),
        out_shape=out_shape,
    )(agg, x, root, bias, gamma, beta, w2)
    return res


_MESH = plsc.VectorSubcoreMesh(core_axis_name="c", subcore_axis_name="s")

_GDN = lax.GatherDimensionNumbers(
    offset_dims=(), collapsed_slice_dims=(0,), start_index_map=(0,))


def _lane_bcast(vec, k):
    idx = jnp.full((16, 1), k, jnp.int32)
    return lax.gather(vec, idx, _GDN, slice_sizes=(1,),
                      mode=lax.GatherScatterMode.PROMISE_IN_BOUNDS)


def _lo(v):
    return lax.bitcast_convert_type(lax.shift_left(v, 16), jnp.float32)


def _hi(v):
    return lax.bitcast_convert_type(v, jnp.float32)


@functools.partial(
    pl.kernel,
    out_type=jax.ShapeDtypeStruct((NC, N_PAD, HID), jnp.float32),
    mesh=_MESH,
    scratch_types=[
        pltpu.VMEM((NBUF, 2, B), jnp.int32),
        pltpu.VMEM((NBUF, B, D_EDGE), jnp.float32),
        pltpu.VMEM((NBUF, B, TCOLS // 2), jnp.int32),
        pltpu.VMEM((B, HID), jnp.float32),
        pltpu.VMEM_SHARED((N, TCOLS // 2), jnp.int32),
        pltpu.VMEM_SHARED((N_PAD, HID), jnp.float32),
        pltpu.SemaphoreType.DMA((NBUF, 4)),
        pltpu.SemaphoreType.DMA((NBUF,)),
        pltpu.SemaphoreType.DMA((NBUF,)),
    ],
    compiler_params=pltpu.CompilerParams(use_tc_tiling_on_sc=False),
)
def _edge_pass(table_hbm, sd_hbm, ef_hbm, zero_hbm, out_hbm,
               sd_v, ef_v, rows_v, msg_v, tbl_sh, agg_sh, gsem, esem, isem):
    c = lax.axis_index("c")
    s = lax.axis_index("s")
    wid = s * NC + c

    pltpu.sync_copy(zero_hbm, agg_sh.at[pl.ds(s * ROWS_PER_TILE,
                                              ROWS_PER_TILE)])
    pltpu.sync_copy(table_hbm.at[pl.ds(s * STG, STG)],
                    tbl_sh.at[pl.ds(s * STG, STG)])

    @pl.when(s == 0)
    def _():
        pltpu.sync_copy(table_hbm.at[pl.ds(NS * STG, N - NS * STG)],
                        tbl_sh.at[pl.ds(NS * STG, N - NS * STG)])

    plsc.subcore_barrier()

    def start_idx(j, slot):
        e0 = (wid * CHUNKS + j) * B
        pltpu.async_copy(sd_hbm.at[:, pl.ds(e0, B)], sd_v.at[slot],
                         isem.at[slot])

    def wait_idx(j, slot):
        e0 = (wid * CHUNKS + j) * B
        pltpu.make_async_copy(sd_hbm.at[:, pl.ds(e0, B)], sd_v.at[slot],
                              isem.at[slot]).wait()

    SPLITS = ((0, 24), (24, 24), (48, 16), (64, 16))

    def start_gather(j, slot):
        e0 = jnp.minimum((wid * CHUNKS + j) * B, E - B)
        pltpu.async_copy(ef_hbm.at[pl.ds(e0, B)], ef_v.at[slot],
                         esem.at[slot])
        for h, (o, n) in enumerate(SPLITS):
            pltpu.async_copy(tbl_sh.at[sd_v.at[slot, 0, pl.ds(o, n)]],
                             rows_v.at[slot, pl.ds(o, n)],
                             gsem.at[slot, h])

    def wait_gather(j, slot):
        e0 = jnp.minimum((wid * CHUNKS + j) * B, E - B)
        pltpu.make_async_copy(ef_hbm.at[pl.ds(e0, B)], ef_v.at[slot],
                              esem.at[slot]).wait()
        for h, (o, n) in enumerate(SPLITS):
            pltpu.make_async_copy(tbl_sh.at[sd_v.at[slot, 0, pl.ds(o, n)]],
                                  rows_v.at[slot, pl.ds(o, n)],
                                  gsem.at[slot, h]).wait()

    def process(j, slot):
        def _one_edge(b):
            efr = ef_v[slot, b, :]
            paccs = [rows_v[slot, b, pl.ds(8 * 16, 16)], None, None, None]
            paccs[0] = _lo(paccs[0])
            for p in range(8):
                pair = rows_v[slot, b, pl.ds(p * 16, 16)]
                lane = p % 4
                term = (_lane_bcast(efr, 2 * p) * _lo(pair)
                        + _lane_bcast(efr, 2 * p + 1) * _hi(pair))
                if paccs[lane] is None:
                    paccs[lane] = term
                else:
                    paccs[lane] = paccs[lane] + term
            msg_v[b, :] = (paccs[0] + paccs[1]) + (paccs[2] + paccs[3])

        def _body(bb, carry2):
            _one_edge(2 * bb)
            _one_edge(2 * bb + 1)
            return carry2

        lax.fori_loop(0, B // 2, _body, 0)
        pltpu.sync_copy(msg_v, agg_sh.at[sd_v.at[slot, 1]], add=True)

    start_idx(0, 0)
    wait_idx(0, 0)
    start_gather(0, 0)
    start_idx(1, 1)

    def outer(jj, carry):
        for slot in range(NBUF):
            j = jj * NBUF + slot
            nslot = (slot + 1) % NBUF

            @pl.when(j + 1 < CHUNKS)
            def _():
                wait_idx(j + 1, nslot)
                start_gather(j + 1, nslot)

            wait_gather(j, slot)
            process(j, slot)

            @pl.when(j + 2 < CHUNKS)
            def _():
                start_idx(j + 2, slot)
        return carry

    lax.fori_loop(0, CHUNKS // NBUF, outer, 0)
    plsc.subcore_barrier()
    pltpu.sync_copy(agg_sh.at[pl.ds(s * ROWS_PER_TILE, ROWS_PER_TILE)],
                    out_hbm.at[c, pl.ds(s * ROWS_PER_TILE, ROWS_PER_TILE)])


_PERM = np.zeros((TCOLS,), np.int32)
for _p in range(PAIRS):
    for _o in range(16):
        _PERM[_p * 16 + _o] = (2 * _p) * 16 + _o
        _PERM[TCOLS // 2 + _p * 16 + _o] = (2 * _p + 1) * 16 + _o


def _pack_wcat(We, be, in_c, out_c):
    Wr = We.reshape(in_c, out_c, D_EDGE).transpose(0, 2, 1)
    Wr = Wr.reshape(in_c, D_EDGE * out_c)
    br = be.reshape(in_c, out_c)
    wcat = jnp.concatenate(
        [Wr, br, jnp.zeros((in_c, 16), jnp.float32)], axis=1)
    return wcat[:, _PERM]


@jax.jit
def kernel(node_features, edge_index, edge_features, batch,
           We1, be1, root1, bias1, gamma1, beta1,
           We2, be2, root2, bias2, gamma2, beta2,
           Wl, bl):
    x = node_features
    src = edge_index[0]
    dst = edge_index[1]

    sd = _tc_sd(edge_index)
    zero_rows = jnp.zeros((ROWS_PER_TILE, HID), jnp.float32)

    wcat1 = _pack_wcat(We1, be1, D_NODE, HID)
    wcat2 = _pack_wcat(We2, be2, HID, HID)

    t1i = _tc_matmul(x, wcat1, pack=True)
    agg1 = _edge_pass(t1i, sd, edge_features, zero_rows)
    h1, t2 = _tc_combine(agg1, x, root1, bias1.reshape(1, HID),
                         gamma1.reshape(1, HID), beta1.reshape(1, HID),
                         wcat2, True)
    agg2 = _edge_pass(t2, sd, edge_features, zero_rows)
    (h2,) = _tc_combine(agg2, h1, root2, bias2.reshape(1, HID),
                        gamma2.reshape(1, HID), beta2.reshape(1, HID),
                        wcat2, False)
    return _tc_matmul_bias(h2, Wl.T, bl.reshape(1, OUT))


# speedup vs baseline: 1.4872x; 1.0012x over previous
import functools

import jax
import jax.numpy as jnp
import numpy as np
from jax import lax
from jax.experimental import pallas as pl
from jax.experimental.pallas import tpu as pltpu
from jax.experimental.pallas import tpu_sc as plsc

N = 10000
E = 320000
D_NODE = 128
D_EDGE = 16
HID = 16
OUT = 128

NC = 2
NS = 16
NW = NC * NS
B = 80
EPT = 10240
E_PAD = NW * EPT
CHUNKS = EPT // B
N_PAD = 10112
ROWS_PER_TILE = N_PAD // NS
PAIRS = 9
TCOLS = PAIRS * 32
NBUF = 2
STG = 624


def _pack_i32(t):
    a = lax.bitcast_convert_type(t[:, :TCOLS // 2].astype(jnp.bfloat16),
                                 jnp.uint16).astype(jnp.int32)
    b = lax.bitcast_convert_type(t[:, TCOLS // 2:].astype(jnp.bfloat16),
                                 jnp.uint16).astype(jnp.int32)
    return lax.bitwise_or(a, lax.shift_left(b, 16))


def _mm_body(pack, x_ref, w_ref, o_ref):
    t = jnp.dot(x_ref[...], w_ref[...], preferred_element_type=jnp.float32)
    o_ref[...] = _pack_i32(t) if pack else t


def _tc_matmul(x, w, pack=False):
    n_out = w.shape[1] // 2 if pack else w.shape[1]
    dt = jnp.int32 if pack else jnp.float32
    return pl.pallas_call(
        functools.partial(_mm_body, pack),
        out_shape=jax.ShapeDtypeStruct((x.shape[0], n_out), dt),
    )(x, w)


def _mmb_body(x_ref, w_ref, b_ref, o_ref):
    o_ref[...] = jnp.dot(x_ref[...], w_ref[...],
                         preferred_element_type=jnp.float32) + b_ref[...]


def _tc_matmul_bias(x, w, b):
    return pl.pallas_call(
        _mmb_body,
        out_shape=jax.ShapeDtypeStruct((x.shape[0], w.shape[1]), jnp.float32),
    )(x, w, b)


def _sd_body(ei_ref, o_ref):
    o_ref[0:1, :E] = ei_ref[0:1, :]
    o_ref[1:2, :E] = ei_ref[1:2, :]
    o_ref[0:1, E:] = jnp.zeros((1, E_PAD - E), jnp.int32)
    o_ref[1:2, E:] = jnp.full((1, E_PAD - E), N, jnp.int32)


def _tc_sd(edge_index):
    return pl.pallas_call(
        _sd_body,
        out_shape=jax.ShapeDtypeStruct((2, E_PAD), jnp.int32),
    )(edge_index)


def _combine_body(make_t2, agg_ref, x_ref, root_ref, bias_ref, gamma_ref,
                  beta_ref, w2_ref, h_ref, t2_ref=None):
    agg = agg_ref[0, :N, :] + agg_ref[1, :N, :]
    h = agg + jnp.dot(x_ref[...], root_ref[...],
                      preferred_element_type=jnp.float32) + bias_ref[...]
    mean = jnp.mean(h, axis=0, keepdims=True)
    d = h - mean
    var = jnp.mean(d * d, axis=0, keepdims=True)
    hn = d * lax.rsqrt(var + 1e-5) * gamma_ref[...] + beta_ref[...]
    hact = jnp.where(hn > 0, hn, 0.01 * hn)
    h_ref[...] = hact
    if make_t2:
        t2_ref[...] = _pack_i32(jnp.dot(hact, w2_ref[...],
                                        preferred_element_type=jnp.float32))


def _tc_combine(agg, x, root, bias, gamma, beta, w2, make_t2):
    out_shape = [jax.ShapeDtypeStruct((N, root.shape[1]), jnp.float32)]
    if make_t2:
        out_shape.append(
            jax.ShapeDtypeStruct((N, w2.shape[1] // 2), jnp.int32))
    res = pl.pallas_call(
        functools.partial(_combine_body, make_t2),
        out_shape=out_shape,
    )(agg, x, root, bias, gamma, beta, w2)
    return res


_MESH = plsc.VectorSubcoreMesh(core_axis_name="c", subcore_axis_name="s")

_GDN = lax.GatherDimensionNumbers(
    offset_dims=(), collapsed_slice_dims=(0,), start_index_map=(0,))


def _lane_bcast(vec, k):
    idx = jnp.full((16, 1), k, jnp.int32)
    return lax.gather(vec, idx, _GDN, slice_sizes=(1,),
                      mode=lax.GatherScatterMode.PROMISE_IN_BOUNDS)


def _lo(v):
    return lax.bitcast_convert_type(lax.shift_left(v, 16), jnp.float32)


def _hi(v):
    return lax.bitcast_convert_type(v, jnp.float32)


@functools.partial(
    pl.kernel,
    out_type=jax.ShapeDtypeStruct((NC, N_PAD, HID), jnp.float32),
    mesh=_MESH,
    scratch_types=[
        pltpu.VMEM((NBUF, 2, B), jnp.int32),
        pltpu.VMEM((NBUF, B, D_EDGE), jnp.float32),
        pltpu.VMEM((NBUF, B, TCOLS // 2), jnp.int32),
        pltpu.VMEM((B, HID), jnp.float32),
        pltpu.VMEM_SHARED((N, TCOLS // 2), jnp.int32),
        pltpu.VMEM_SHARED((N_PAD, HID), jnp.float32),
        pltpu.SemaphoreType.DMA((NBUF, 4)),
        pltpu.SemaphoreType.DMA((NBUF,)),
        pltpu.SemaphoreType.DMA((NBUF,)),
    ],
    compiler_params=pltpu.CompilerParams(use_tc_tiling_on_sc=False),
)
def _edge_pass(table_hbm, sd_hbm, ef_hbm, zero_hbm, out_hbm,
               sd_v, ef_v, rows_v, msg_v, tbl_sh, agg_sh, gsem, esem, isem):
    c = lax.axis_index("c")
    s = lax.axis_index("s")
    wid = s * NC + c

    pltpu.sync_copy(zero_hbm, agg_sh.at[pl.ds(s * ROWS_PER_TILE,
                                              ROWS_PER_TILE)])
    pltpu.sync_copy(table_hbm.at[pl.ds(s * STG, STG)],
                    tbl_sh.at[pl.ds(s * STG, STG)])

    @pl.when(s == 0)
    def _():
        pltpu.sync_copy(table_hbm.at[pl.ds(NS * STG, N - NS * STG)],
                        tbl_sh.at[pl.ds(NS * STG, N - NS * STG)])

    plsc.subcore_barrier()

    def start_idx(j, slot):
        e0 = (wid * CHUNKS + j) * B
        pltpu.async_copy(sd_hbm.at[:, pl.ds(e0, B)], sd_v.at[slot],
                         isem.at[slot])

    def wait_idx(j, slot):
        e0 = (wid * CHUNKS + j) * B
        pltpu.make_async_copy(sd_hbm.at[:, pl.ds(e0, B)], sd_v.at[slot],
                              isem.at[slot]).wait()

    SPLITS = ((0, 24), (24, 24), (48, 16), (64, 16))

    def start_gather(j, slot):
        e0 = jnp.minimum((wid * CHUNKS + j) * B, E - B)
        pltpu.async_copy(ef_hbm.at[pl.ds(e0, B)], ef_v.at[slot],
                         esem.at[slot])
        for h, (o, n) in enumerate(SPLITS):
            pltpu.async_copy(tbl_sh.at[sd_v.at[slot, 0, pl.ds(o, n)]],
                             rows_v.at[slot, pl.ds(o, n)],
                             gsem.at[slot, h])

    def wait_gather(j, slot):
        e0 = jnp.minimum((wid * CHUNKS + j) * B, E - B)
        pltpu.make_async_copy(ef_hbm.at[pl.ds(e0, B)], ef_v.at[slot],
                              esem.at[slot]).wait()
        for h, (o, n) in enumerate(SPLITS):
            pltpu.make_async_copy(tbl_sh.at[sd_v.at[slot, 0, pl.ds(o, n)]],
                                  rows_v.at[slot, pl.ds(o, n)],
                                  gsem.at[slot, h]).wait()

    def process(j, slot):
        def _one_edge(b):
            efr = ef_v[slot, b, :]
            paccs = [rows_v[slot, b, pl.ds(8 * 16, 16)], None, None, None]
            paccs[0] = _lo(paccs[0])
            for p in range(8):
                pair = rows_v[slot, b, pl.ds(p * 16, 16)]
                lane = p % 4
                term = (_lane_bcast(efr, 2 * p) * _lo(pair)
                        + _lane_bcast(efr, 2 * p + 1) * _hi(pair))
                if paccs[lane] is None:
                    paccs[lane] = term
                else:
                    paccs[lane] = paccs[lane] + term
            msg_v[b, :] = (paccs[0] + paccs[1]) + (paccs[2] + paccs[3])

        def _body(bb, carry2):
            _one_edge(2 * bb)
            _one_edge(2 * bb + 1)
            return carry2

        lax.fori_loop(0, B // 2, _body, 0)
        pltpu.sync_copy(msg_v, agg_sh.at[sd_v.at[slot, 1]], add=True)

    start_idx(0, 0)
    wait_idx(0, 0)
    start_gather(0, 0)
    start_idx(1, 1)

    def outer(jj, carry):
        for slot in range(NBUF):
            j = jj * NBUF + slot
            nslot = (slot + 1) % NBUF

            @pl.when(j + 1 < CHUNKS)
            def _():
                wait_idx(j + 1, nslot)
                start_gather(j + 1, nslot)

            wait_gather(j, slot)
            process(j, slot)

            @pl.when(j + 2 < CHUNKS)
            def _():
                start_idx(j + 2, slot)
        return carry

    lax.fori_loop(0, CHUNKS // NBUF, outer, 0)
    plsc.subcore_barrier()
    pltpu.sync_copy(agg_sh.at[pl.ds(s * ROWS_PER_TILE, ROWS_PER_TILE)],
                    out_hbm.at[c, pl.ds(s * ROWS_PER_TILE, ROWS_PER_TILE)])


_PERM = np.zeros((TCOLS,), np.int32)
for _p in range(PAIRS):
    for _o in range(16):
        _PERM[_p * 16 + _o] = (2 * _p) * 16 + _o
        _PERM[TCOLS // 2 + _p * 16 + _o] = (2 * _p + 1) * 16 + _o


def _pack_wcat(We, be, in_c, out_c):
    Wr = We.reshape(in_c, out_c, D_EDGE).transpose(0, 2, 1)
    Wr = Wr.reshape(in_c, D_EDGE * out_c)
    br = be.reshape(in_c, out_c)
    wcat = jnp.concatenate(
        [Wr, br, jnp.zeros((in_c, 16), jnp.float32)], axis=1)
    return wcat[:, _PERM]


@jax.jit
def kernel(node_features, edge_index, edge_features, batch,
           We1, be1, root1, bias1, gamma1, beta1,
           We2, be2, root2, bias2, gamma2, beta2,
           Wl, bl):
    x = node_features

    sd = _tc_sd(edge_index)
    zero_rows = jnp.zeros((ROWS_PER_TILE, HID), jnp.float32)

    wcat1 = _pack_wcat(We1, be1, D_NODE, HID)
    wcat2 = _pack_wcat(We2, be2, HID, HID)

    t1i = _tc_matmul(x, wcat1, pack=True)
    agg1 = _edge_pass(t1i, sd, edge_features, zero_rows)
    h1, t2 = _tc_combine(agg1, x, root1, bias1.reshape(1, HID),
                         gamma1.reshape(1, HID), beta1.reshape(1, HID),
                         wcat2, True)
    agg2 = _edge_pass(t2, sd, edge_features, zero_rows)
    (h2,) = _tc_combine(agg2, h1, root2, bias2.reshape(1, HID),
                        gamma2.reshape(1, HID), beta2.reshape(1, HID),
                        wcat2, False)
    return _tc_matmul_bias(h2, Wl.T, bl.reshape(1, OUT))
